# baseline (speedup 1.0000x reference)
import functools

import jax
import jax.numpy as jnp
from jax import lax
from jax.experimental import pallas as pl
from jax.experimental.pallas import tpu as pltpu
from jax.experimental.pallas import tpu_sc as plsc

_V = 1_000_000
_D = 64
_BATCH = 4096
_SEQ = 200
_B = _BATCH * _SEQ
_NW = 32
_LANES = 128
_NBLK = _B // _LANES
_BLK_PER_W = _NBLK // _NW
_NPAIR = _BLK_PER_W // 2
_BT = _BATCH // _LANES
_SCALE = 8.0
_W = 384
_NWIN = _V // _W
_VMAIN = _NWIN * _W
_VTAIL = _V - _VMAIN

_mesh = plsc.VectorSubcoreMesh(core_axis_name="c", subcore_axis_name="s")


def _transpose_window(tin, tout, width, lane):
    @plsc.parallel_loop(0, width // 16)
    def q_body(q):
        dv16 = q * 16 + lane
        rowv = dv16 >> 1
        colb = (dv16 & 1) * _D

        @plsc.parallel_loop(0, _D, unroll=8)
        def j_body(j):
            dvec = (lane + j) & (_D - 1)
            vals = plsc.load_gather(tin, [dvec, dv16])
            plsc.store_scatter(tout, [rowv, colb + dvec], vals)


@functools.partial(
    pl.kernel,
    out_type=jax.ShapeDtypeStruct((_V // 2, 2 * _D), jnp.float32),
    mesh=_mesh,
    compiler_params=pltpu.CompilerParams(needs_layout_passes=False),
    scratch_types=[
        pltpu.VMEM((_D, _W), jnp.float32),
        pltpu.VMEM((_D, _W), jnp.float32),
        pltpu.VMEM((_W // 2, 2 * _D), jnp.float32),
        pltpu.VMEM((_W // 2, 2 * _D), jnp.float32),
        pltpu.VMEM((_D, _VTAIL), jnp.float32),
        pltpu.VMEM((_VTAIL // 2, 2 * _D), jnp.float32),
        pltpu.SemaphoreType.DMA,
        pltpu.SemaphoreType.DMA,
        pltpu.SemaphoreType.DMA,
        pltpu.SemaphoreType.DMA,
    ],
)
def _pack_table(tt_hbm, ttail_hbm, out_hbm, tin_a, tin_b, tout_a, tout_b,
                tin_t, tout_t, gsa, gsb, osa, osb):
    wid = lax.axis_index("s") * 2 + lax.axis_index("c")
    lane = lax.iota(jnp.int32, 16)
    nk = _NWIN // _NW + 1

    def start_in(tin, w, sem):
        pltpu.async_copy(tt_hbm.at[:, pl.ds(w * _W, _W)], tin, sem)

    def wait_in(tin, sem):
        pltpu.make_async_copy(tt_hbm.at[:, pl.ds(0, _W)], tin, sem).wait()

    def start_out(tout, w, sem):
        pltpu.async_copy(
            tout, out_hbm.at[pl.ds(w * (_W // 2), _W // 2)], sem)

    def wait_out(tout, sem):
        pltpu.make_async_copy(
            tout, out_hbm.at[pl.ds(0, _W // 2)], sem).wait()

    def guarded(w, fn):
        pl.when(w < _NWIN)(fn)

    w0 = wid
    guarded(w0, lambda: start_in(tin_a, w0, gsa))

    def k_body(h, carry):
        wa = wid + (2 * h) * _NW
        wb = wid + (2 * h + 1) * _NW

        def do_a():
            wait_in(tin_a, gsa)
            pl.when(h > 0)(lambda: wait_out(tout_a, osa))
            _transpose_window(tin_a, tout_a, _W, lane)
            start_out(tout_a, wa, osa)

        guarded(wb, lambda: start_in(tin_b, wb, gsb))
        guarded(wa, do_a)
        guarded(wid + (2 * h + 2) * _NW,
                lambda: start_in(tin_a, wid + (2 * h + 2) * _NW, gsa))

        def do_b():
            wait_in(tin_b, gsb)
            pl.when(h > 0)(lambda: wait_out(tout_b, osb))
            _transpose_window(tin_b, tout_b, _W, lane)
            start_out(tout_b, wb, osb)

        guarded(wb, do_b)
        return carry

    lax.fori_loop(0, (nk + 1) // 2, k_body, 0)
    guarded(w0, lambda: wait_out(tout_a, osa))
    guarded(wid + _NW, lambda: wait_out(tout_b, osb))

    @pl.when(wid == 0)
    def _():
        pltpu.sync_copy(ttail_hbm, tin_t)
        _transpose_window(tin_t, tout_t, _VTAIL, lane)
        pltpu.sync_copy(
            tout_t, out_hbm.at[pl.ds(_VMAIN // 2, _VTAIL // 2)])


@functools.partial(
    pl.kernel,
    out_type=jax.ShapeDtypeStruct((_SEQ, _D // 8, _BT, 8, _LANES),
                                  jnp.float32),
    mesh=_mesh,
    compiler_params=pltpu.CompilerParams(needs_layout_passes=False,
                                         use_tc_tiling_on_sc=False),
    scratch_types=[
        pltpu.VMEM((_BLK_PER_W, _LANES), jnp.int32),
        pltpu.VMEM((_LANES,), jnp.int32),
        pltpu.VMEM((_LANES,), jnp.int32),
        pltpu.VMEM((_LANES, _D), jnp.float32),
        pltpu.VMEM((_LANES, _D), jnp.float32),
        pltpu.VMEM((_D, _LANES), jnp.float32),
        pltpu.VMEM((_D, _LANES), jnp.float32),
        pltpu.SemaphoreType.DMA,
        pltpu.SemaphoreType.DMA,
        pltpu.SemaphoreType.DMA,
        pltpu.SemaphoreType.DMA,
    ],
)
def _gather(xt_hbm, table_hbm, out_hbm, idx_v, gidx_a, gidx_b, rows_a,
            rows_b, outv_a, outv_b, gsa, gsb, osa, osb):
    wid = lax.axis_index("s") * 2 + lax.axis_index("c")
    rbase = wid * _BLK_PER_W
    pltpu.sync_copy(xt_hbm.at[pl.ds(rbase, _BLK_PER_W)], idx_v)
    lane = lax.iota(jnp.int32, 16)

    def start_gather(gidx, rows_v, i, sem):
        @plsc.parallel_loop(0, _LANES // 16)
        def cp(q):
            gidx[pl.ds(q * 16, 16)] = idx_v[i, pl.ds(q * 16, 16)]

        pltpu.async_copy(table_hbm.at[gidx], rows_v, sem)

    def wait_gather(gidx, rows_v, sem):
        pltpu.make_async_copy(table_hbm.at[gidx], rows_v, sem).wait()

    def transpose_block(rows_v, out_v, i):
        @plsc.parallel_loop(0, _LANES // 16)
        def q_body(q):
            tok16 = q * 16 + lane

            @plsc.parallel_loop(0, _D, unroll=8)
            def j_body(j):
                cvec = (lane + j) & (_D - 1)
                vals = plsc.load_gather(rows_v, [tok16, cvec])
                plsc.store_scatter(out_v, [cvec, tok16], vals * _SCALE)

    def start_out(out_v, i, sem):
        r = rbase + i
        l = r // _BT
        bt = r % _BT
        for d1 in range(_D // 8):
            pltpu.async_copy(out_v.at[pl.ds(d1 * 8, 8)],
                             out_hbm.at[l, d1, bt], sem)

    def wait_out(out_v, sem):
        for d1 in range(_D // 8):
            pltpu.make_async_copy(out_v.at[pl.ds(d1 * 8, 8)],
                                  out_hbm.at[0, d1, 0], sem).wait()

    start_gather(gidx_a, rows_a, 0, gsa)

    def pair_body(h, carry):
        e = 2 * h
        o = e + 1
        start_gather(gidx_b, rows_b, o, gsb)
        wait_gather(gidx_a, rows_a, gsa)
        pl.when(h > 0)(lambda: wait_out(outv_a, osa))
        transpose_block(rows_a, outv_a, e)
        start_out(outv_a, e, osa)
        pl.when(h < _NPAIR - 1)(
            lambda: start_gather(gidx_a, rows_a, e + 2, gsa))
        wait_gather(gidx_b, rows_b, gsb)
        pl.when(h > 0)(lambda: wait_out(outv_b, osb))
        transpose_block(rows_b, outv_b, o)
        start_out(outv_b, o, osb)
        return carry

    lax.fori_loop(0, _NPAIR, pair_body, 0)
    wait_out(outv_a, osa)
    wait_out(outv_b, osb)


def kernel(x, table):
    xt = x.T.reshape(_NBLK, _LANES).astype(jnp.int32)
    tt = table.T
    ttail = table[_VMAIN:, :].T
    tpack = _pack_table(tt, ttail)
    trows = tpack.reshape(_V, _D)
    out5 = _gather(xt, trows)
    return out5.transpose(2, 4, 0, 1, 3).reshape(_BATCH, _SEQ, _D)


# speedup vs baseline: 1.0582x; 1.0582x over previous
import functools

import jax
import jax.numpy as jnp
from jax import lax
from jax.experimental import pallas as pl
from jax.experimental.pallas import tpu as pltpu
from jax.experimental.pallas import tpu_sc as plsc

_V = 1_000_000
_D = 64
_BATCH = 4096
_SEQ = 200
_B = _BATCH * _SEQ
_NW = 32
_LANES = 128
_NBLK = _B // _LANES
_BLK_PER_W = _NBLK // _NW
_NPAIR = _BLK_PER_W // 2
_BT = _BATCH // _LANES
_SCALE = 8.0
_W = 256
_NWIN = _V // _W
_VMAIN = _NWIN * _W
_VTAIL = _V - _VMAIN

_mesh = plsc.VectorSubcoreMesh(core_axis_name="c", subcore_axis_name="s")


def _transpose_window(tin, tout, width, lane):
    @plsc.parallel_loop(0, width // 16)
    def q_body(q):
        dv16 = q * 16 + lane
        rowv = dv16 >> 1
        colb = (dv16 & 1) * _D

        @plsc.parallel_loop(0, _D, unroll=8)
        def j_body(j):
            dvec = (lane + j) & (_D - 1)
            vals = plsc.load_gather(tin, [dvec, dv16])
            plsc.store_scatter(tout, [rowv, colb + dvec], vals)


@functools.partial(
    pl.kernel,
    out_type=jax.ShapeDtypeStruct((_V // 2, 2 * _D), jnp.float32),
    mesh=_mesh,
    compiler_params=pltpu.CompilerParams(needs_layout_passes=False),
    scratch_types=[
        pltpu.VMEM((_D, _W), jnp.float32),
        pltpu.VMEM((_D, _W), jnp.float32),
        pltpu.VMEM((_W // 2, 2 * _D), jnp.float32),
        pltpu.VMEM((_W // 2, 2 * _D), jnp.float32),
        pltpu.VMEM((_D, _VTAIL), jnp.float32),
        pltpu.VMEM((_VTAIL // 2, 2 * _D), jnp.float32),
        pltpu.SemaphoreType.DMA,
        pltpu.SemaphoreType.DMA,
        pltpu.SemaphoreType.DMA,
        pltpu.SemaphoreType.DMA,
    ],
)
def _pack_table(tt_hbm, ttail_hbm, out_hbm, tin_a, tin_b, tout_a, tout_b,
                tin_t, tout_t, gsa, gsb, osa, osb):
    wid = lax.axis_index("s") * 2 + lax.axis_index("c")
    lane = lax.iota(jnp.int32, 16)
    nk = _NWIN // _NW + 1

    def start_in(tin, w, sem):
        pltpu.async_copy(tt_hbm.at[:, pl.ds(w * _W, _W)], tin, sem)

    def wait_in(tin, sem):
        pltpu.make_async_copy(tt_hbm.at[:, pl.ds(0, _W)], tin, sem).wait()

    def start_out(tout, w, sem):
        pltpu.async_copy(
            tout, out_hbm.at[pl.ds(w * (_W // 2), _W // 2)], sem)

    def wait_out(tout, sem):
        pltpu.make_async_copy(
            tout, out_hbm.at[pl.ds(0, _W // 2)], sem).wait()

    def guarded(w, fn):
        pl.when(w < _NWIN)(fn)

    w0 = wid
    guarded(w0, lambda: start_in(tin_a, w0, gsa))

    def k_body(h, carry):
        wa = wid + (2 * h) * _NW
        wb = wid + (2 * h + 1) * _NW

        def do_a():
            wait_in(tin_a, gsa)
            pl.when(h > 0)(lambda: wait_out(tout_a, osa))
            _transpose_window(tin_a, tout_a, _W, lane)
            start_out(tout_a, wa, osa)

        guarded(wb, lambda: start_in(tin_b, wb, gsb))
        guarded(wa, do_a)
        guarded(wid + (2 * h + 2) * _NW,
                lambda: start_in(tin_a, wid + (2 * h + 2) * _NW, gsa))

        def do_b():
            wait_in(tin_b, gsb)
            pl.when(h > 0)(lambda: wait_out(tout_b, osb))
            _transpose_window(tin_b, tout_b, _W, lane)
            start_out(tout_b, wb, osb)

        guarded(wb, do_b)
        return carry

    lax.fori_loop(0, (nk + 1) // 2, k_body, 0)
    guarded(w0, lambda: wait_out(tout_a, osa))
    guarded(wid + _NW, lambda: wait_out(tout_b, osb))

    @pl.when(wid == 0)
    def _():
        pltpu.sync_copy(ttail_hbm, tin_t)
        _transpose_window(tin_t, tout_t, _VTAIL, lane)
        pltpu.sync_copy(
            tout_t, out_hbm.at[pl.ds(_VMAIN // 2, _VTAIL // 2)])


@functools.partial(
    pl.kernel,
    out_type=jax.ShapeDtypeStruct((_SEQ, _D // 8, _BT, 8, _LANES),
                                  jnp.float32),
    mesh=_mesh,
    compiler_params=pltpu.CompilerParams(needs_layout_passes=False,
                                         use_tc_tiling_on_sc=False),
    scratch_types=[
        pltpu.VMEM((_BLK_PER_W, _LANES), jnp.int32),
        pltpu.VMEM((_LANES,), jnp.int32),
        pltpu.VMEM((_LANES,), jnp.int32),
        pltpu.VMEM((_LANES, _D), jnp.float32),
        pltpu.VMEM((_LANES, _D), jnp.float32),
        pltpu.VMEM((_D, _LANES), jnp.float32),
        pltpu.VMEM((_D, _LANES), jnp.float32),
        pltpu.SemaphoreType.DMA,
        pltpu.SemaphoreType.DMA,
        pltpu.SemaphoreType.DMA,
        pltpu.SemaphoreType.DMA,
    ],
)
def _gather(xt_hbm, table_hbm, out_hbm, idx_v, gidx_a, gidx_b, rows_a,
            rows_b, outv_a, outv_b, gsa, gsb, osa, osb):
    wid = lax.axis_index("s") * 2 + lax.axis_index("c")
    rbase = wid * _BLK_PER_W
    pltpu.sync_copy(xt_hbm.at[pl.ds(rbase, _BLK_PER_W)], idx_v)
    lane = lax.iota(jnp.int32, 16)

    def start_gather(gidx, rows_v, i, sem):
        @plsc.parallel_loop(0, _LANES // 16)
        def cp(q):
            gidx[pl.ds(q * 16, 16)] = idx_v[i, pl.ds(q * 16, 16)]

        pltpu.async_copy(table_hbm.at[gidx], rows_v, sem)

    def wait_gather(gidx, rows_v, sem):
        pltpu.make_async_copy(table_hbm.at[gidx], rows_v, sem).wait()

    def transpose_block(rows_v, out_v, i):
        @plsc.parallel_loop(0, _LANES // 16)
        def q_body(q):
            tok16 = q * 16 + lane

            @plsc.parallel_loop(0, _D, unroll=8)
            def j_body(j):
                cvec = (lane + j) & (_D - 1)
                vals = plsc.load_gather(rows_v, [tok16, cvec])
                plsc.store_scatter(out_v, [cvec, tok16], vals * _SCALE)

    def start_out(out_v, i, sem):
        r = rbase + i
        l = r // _BT
        bt = r % _BT
        for d1 in range(_D // 8):
            pltpu.async_copy(out_v.at[pl.ds(d1 * 8, 8)],
                             out_hbm.at[l, d1, bt], sem)

    def wait_out(out_v, sem):
        for d1 in range(_D // 8):
            pltpu.make_async_copy(out_v.at[pl.ds(d1 * 8, 8)],
                                  out_hbm.at[0, d1, 0], sem).wait()

    start_gather(gidx_a, rows_a, 0, gsa)

    def pair_body(h, carry):
        e = 2 * h
        o = e + 1
        start_gather(gidx_b, rows_b, o, gsb)
        wait_gather(gidx_a, rows_a, gsa)
        pl.when(h > 0)(lambda: wait_out(outv_a, osa))
        transpose_block(rows_a, outv_a, e)
        start_out(outv_a, e, osa)
        pl.when(h < _NPAIR - 1)(
            lambda: start_gather(gidx_a, rows_a, e + 2, gsa))
        wait_gather(gidx_b, rows_b, gsb)
        pl.when(h > 0)(lambda: wait_out(outv_b, osb))
        transpose_block(rows_b, outv_b, o)
        start_out(outv_b, o, osb)
        return carry

    lax.fori_loop(0, _NPAIR, pair_body, 0)
    wait_out(outv_a, osa)
    wait_out(outv_b, osb)


def kernel(x, table):
    xt = x.T.reshape(_NBLK, _LANES).astype(jnp.int32)
    tt = table.T
    ttail = table[_VMAIN:, :].T
    tpack = _pack_table(tt, ttail)
    trows = tpack.reshape(_V, _D)
    out5 = _gather(xt, trows)
    return out5.transpose(2, 4, 0, 1, 3).reshape(_BATCH, _SEQ, _D)
